# baseline (speedup 1.0000x reference)
import functools

import jax
import jax.numpy as jnp
from jax import lax
from jax.experimental import pallas as pl
from jax.experimental.pallas import tpu as pltpu
from jax.experimental.pallas import tpu_sc as plsc

N = 10000
E = 320000
D = 128
DH = 64
NPAD = 10240
NC, NS = 2, 16
K = 80
CPT = E // (NS * K)
HALF = CPT // 2
NACC = N
RPT = NACC // NS
JS = [K] * 7 + [RPT - 7 * K]
NBUF = 4
BN = 2048
DEGW = 8
NT = NC * NS
CPTD = E // (NT * K)


def _sc_agg_body(hf, srch, dsth, eth, agga, aggb, dego,
                 srcv, dstv, gidxv, b0, b1, b2, b3, degv, acc,
                 s0, s1, s2, s3):
    bufs = ((b0, s0), (b1, s1), (b2, s2), (b3, s3))
    c = lax.axis_index("c")
    s = lax.axis_index("s")
    wid = s * NC + c

    z16 = jnp.zeros((16,), jnp.float32)

    def _bz(r, carry):
        for i in range(DH // 16):
            b0[r, pl.ds(i * 16, 16)] = z16
        return carry

    lax.fori_loop(0, K, _bz, 0)
    rb = s * RPT
    off = 0
    for sz in JS:
        pltpu.sync_copy(b0.at[pl.ds(0, sz)], acc.at[pl.ds(rb + off, sz)])
        off += sz

    def _dz(i, carry):
        degv[pl.ds(i * 16, 16)] = z16
        return carry

    lax.fori_loop(0, NPAD // 16, _dz, 0)
    plsc.subcore_barrier()

    pltpu.sync_copy(srch.at[s], srcv)
    pltpu.sync_copy(dsth.at[s], dstv)
    pltpu.sync_copy(eth.at[s], gidxv)

    base = c * (9 * NPAD)

    def _idx(g, carry):
        for i in range(K // 16):
            sl = pl.ds(i * 16, 16)
            gidxv[g, sl] = gidxv[g, sl] * NPAD + srcv[g, sl] + base
        return carry

    lax.fori_loop(0, CPT, _idx, 0)

    def _start(g, buf, sem):
        pltpu.async_copy(hf.at[gidxv.at[g]], buf, sem)

    def _wait(buf, sem):
        pltpu.make_async_copy(hf.at[pl.ds(0, K)], buf, sem).wait()

    ones16 = jnp.full((16,), 1.0, jnp.float32)

    def _scat(g, buf):
        pltpu.sync_copy(buf, acc.at[dstv.at[g]], add=True)
        for i in range(K // 16):
            plsc.addupdate_scatter(degv, [dstv[g, pl.ds(i * 16, 16)]],
                                   ones16)

    for b, (buf, sem) in enumerate(bufs):
        _start(b, buf, sem)

    def _group(j, carry):
        for b, (buf, sem) in enumerate(bufs):
            g = NBUF * j + b
            _wait(buf, sem)
            _scat(g, buf)

            @pl.when(g + NBUF < CPT)
            def _():
                _start(g + NBUF, buf, sem)

        return carry

    lax.fori_loop(0, CPT // NBUF, _group, 0)
    for b, (buf, sem) in enumerate(bufs):
        g = NBUF * (CPT // NBUF) + b
        if g < CPT:
            _wait(buf, sem)
            _scat(g, buf)
    plsc.subcore_barrier()

    off = 0
    for sz in JS:
        r0 = rb + off

        @pl.when(c == 0)
        def _():
            pltpu.sync_copy(acc.at[pl.ds(r0, sz)], agga.at[pl.ds(r0, sz)])

        @pl.when(c == 1)
        def _():
            pltpu.sync_copy(acc.at[pl.ds(r0, sz)], aggb.at[pl.ds(r0, sz)])

        off += sz

    pltpu.sync_copy(degv, dego.at[pl.ds(wid * NPAD, NPAD)])


_sc_agg = pl.kernel(
    _sc_agg_body,
    out_type=(jax.ShapeDtypeStruct((NPAD, DH), jnp.float32),
              jax.ShapeDtypeStruct((NPAD, DH), jnp.float32),
              jax.ShapeDtypeStruct((NT * NPAD,), jnp.float32)),
    mesh=plsc.VectorSubcoreMesh(core_axis_name="c", subcore_axis_name="s",
                                num_cores=NC, num_subcores=NS),
    scratch_types=[
        pltpu.VMEM((CPT, K), jnp.int32),
        pltpu.VMEM((CPT, K), jnp.int32),
        pltpu.VMEM((CPT, K), jnp.int32),
    ] + [pltpu.VMEM((K, DH), jnp.float32) for _ in range(NBUF)]
      + [pltpu.VMEM((NPAD,), jnp.float32)]
      + [pltpu.VMEM_SHARED((NACC, DH), jnp.float32)]
      + [pltpu.SemaphoreType.DMA for _ in range(NBUF)],
    compiler_params=pltpu.CompilerParams(use_tc_tiling_on_sc=False,
                                         needs_layout_passes=False),
)


def _mm_body(x_ref, w_ref, o_ref):
    o_ref[0, 0] = jnp.dot(x_ref[...], w_ref[0, 0],
                          preferred_element_type=jnp.float32)


def _mm(xp, w_all):
    return pl.pallas_call(
        _mm_body,
        grid=(2, NPAD // BN, 9),
        in_specs=[pl.BlockSpec((BN, D), lambda h, nb, r: (nb, 0)),
                  pl.BlockSpec((1, 1, D, DH), lambda h, nb, r: (h, r, 0, 0))],
        out_specs=pl.BlockSpec((1, 1, BN, DH), lambda h, nb, r: (h, r, nb, 0)),
        out_shape=jax.ShapeDtypeStruct((2, 9, NPAD, DH), jnp.float32),
    )(xp, w_all)


def _combine_body(aa_ref, ab_ref, deg_ref, ra_ref, rb_ref, b_ref, o_ref,
                  *, act):
    degv = jnp.sum(deg_ref[...], axis=0) * 0.5
    inv = 1.0 / jnp.maximum(degv, 1.0)
    ha = aa_ref[...] * inv[:, None] + ra_ref[...]
    hb = ab_ref[...] * inv[:, None] + rb_ref[...]
    h = jnp.concatenate([ha, hb], axis=1) + b_ref[...]
    o_ref[...] = jnp.maximum(h, 0.0) if act else h


def _combine(agga, aggb, deg, roota, rootb, b2d, act):
    half = pl.BlockSpec((BN, DH), lambda nb: (nb, 0))
    return pl.pallas_call(
        functools.partial(_combine_body, act=act),
        grid=(NPAD // BN,),
        in_specs=[half, half,
                  pl.BlockSpec((NT, BN), lambda nb: (0, nb)),
                  half, half,
                  pl.BlockSpec((1, D), lambda nb: (0, 0))],
        out_specs=pl.BlockSpec((BN, D), lambda nb: (nb, 0)),
        out_shape=jax.ShapeDtypeStruct((NPAD, D), jnp.float32),
    )(agga, aggb, deg, roota, rootb, b2d)


def _layer(xp, w_all, b, src2, dst2, et2, deg_in, act):
    hf3 = _mm(xp, w_all)
    hf = hf3.reshape(2 * 9 * NPAD, DH)
    agga, aggb, dego = _sc_agg(hf, src2, dst2, et2)
    deg = dego.reshape(NT, NPAD) if deg_in is None else deg_in
    h = _combine(agga, aggb, deg, hf3[0, 8], hf3[1, 8], b.reshape(1, D), act)
    return h, deg


def kernel(x, edge_index, edge_type, W_rel1, W_root1, b1, W_rel2, W_root2, b2):
    f32 = jnp.float32
    src2 = edge_index[0].astype(jnp.int32).reshape(NS, CPT, K)
    dst2 = edge_index[1].astype(jnp.int32).reshape(NS, CPT, K)
    et2 = edge_type.astype(jnp.int32).reshape(NS, CPT, K)
    xp = jnp.pad(x.astype(f32), ((0, NPAD - N), (0, 0)))
    w_all1 = jnp.concatenate([W_rel1, W_root1[None]], axis=0).astype(f32)
    w_all2 = jnp.concatenate([W_rel2, W_root2[None]], axis=0).astype(f32)
    w_all1 = w_all1.reshape(9, D, 2, DH).transpose(2, 0, 1, 3)
    w_all2 = w_all2.reshape(9, D, 2, DH).transpose(2, 0, 1, 3)
    h, deg = _layer(xp, w_all1, b1, src2, dst2, et2, None, True)
    out, _ = _layer(h, w_all2, b2, src2, dst2, et2, deg, False)
    return out[:N]
